# baseline (speedup 1.0000x reference)
import functools

import jax
import jax.numpy as jnp
from jax import lax
from jax.experimental import pallas as pl
from jax.experimental.pallas import tpu as pltpu
from jax.experimental.pallas import tpu_sc as plsc

NC = 2
NS = 16
NW = NC * NS

N = 10000
D = 128
E = 320000

B3 = 128
C3 = 80
SW = 16
C3_0 = 128
C3_1 = 32
EP = NW * C3 * B3
RPT = 632
NACC = NS * RPT

_mesh = plsc.VectorSubcoreMesh(core_axis_name="c", subcore_axis_name="s")
_sc_params = pltpu.CompilerParams(needs_layout_passes=False)


EW1 = E // NW


@functools.partial(
    pl.kernel,
    out_type=(
        jax.ShapeDtypeStruct((NW, N), jnp.float32),
        jax.ShapeDtypeStruct((NW, N), jnp.float32),
    ),
    mesh=_mesh,
    scratch_types=[
        pltpu.VMEM((EW1,), jnp.int32),
        pltpu.VMEM((EW1,), jnp.int32),
        pltpu.VMEM((N,), jnp.float32),
        pltpu.VMEM((N,), jnp.float32),
    ],
    compiler_params=_sc_params,
)
def _sc_bincount(row_ref, col_ref, cnt_out, self_out, rowv, colv, cntl, selfl):
    c = lax.axis_index("c")
    s = lax.axis_index("s")
    w = s * NC + c
    base = w * EW1
    pltpu.sync_copy(row_ref.at[pl.ds(base, EW1)], rowv)
    pltpu.sync_copy(col_ref.at[pl.ds(base, EW1)], colv)

    zeros16 = jnp.zeros((16,), jnp.float32)

    def zbody(i, carry):
        cntl[pl.ds(i * 16, 16)] = zeros16
        selfl[pl.ds(i * 16, 16)] = zeros16
        return carry

    lax.fori_loop(0, N // 16, zbody, 0)

    ones16 = jnp.ones((16,), jnp.float32)

    def body(i, carry):
        r = rowv[pl.ds(i * 16, 16)]
        cc = colv[pl.ds(i * 16, 16)]
        plsc.addupdate_scatter(cntl, [r], ones16)
        plsc.addupdate_scatter(
            selfl, [r], jnp.where(r == cc, 1.0, 0.0).astype(jnp.float32)
        )
        return carry

    lax.fori_loop(0, EW1 // 16, body, 0)

    pltpu.sync_copy(cntl, cnt_out.at[w])
    pltpu.sync_copy(selfl, self_out.at[w])


R2 = 2000


def _tc_transform_body(x_ref, m_ref, w_ref, cnt_ref, self_ref, y_ref):
    xb = x_ref[...]
    ox = xb
    m = m_ref[...]
    for i in range(7):
        ox = ox + jnp.dot(
            xb, w_ref[i], preferred_element_type=jnp.float32
        ) * m[:, i + 1 : i + 2]
    cnt = jnp.sum(cnt_ref[...], axis=1)
    selfc = jnp.sum(self_ref[...], axis=1)
    loop_w = jnp.where(selfc == 0.0, 1.0, 0.0)
    deg = cnt + loop_w
    dis = lax.rsqrt(deg)
    dis = jnp.where(jnp.isinf(dis), 0.0, dis)
    y_ref[...] = dis[:, None] * ox


def _tc_transform(x, masks, weight_id, cnt_t, self_t):
    return pl.pallas_call(
        _tc_transform_body,
        grid=(N // R2,),
        in_specs=[
            pl.BlockSpec((R2, D), lambda i: (i, 0)),
            pl.BlockSpec((R2, 8), lambda i: (i, 0)),
            pl.BlockSpec((7, D, D), lambda i: (0, 0, 0)),
            pl.BlockSpec((R2, NW), lambda i: (i, 0)),
            pl.BlockSpec((R2, NW), lambda i: (i, 0)),
        ],
        out_specs=pl.BlockSpec((R2, D), lambda i: (i, 0)),
        out_shape=jax.ShapeDtypeStruct((N, D), jnp.float32),
    )(x, masks, weight_id, cnt_t, self_t)


C3W = NW * C3 // NS
_mesh1 = plsc.VectorSubcoreMesh(
    core_axis_name="c", subcore_axis_name="s", num_cores=1
)


@functools.partial(
    pl.kernel,
    out_type=jax.ShapeDtypeStruct((NACC, D), jnp.float32),
    mesh=_mesh1,
    scratch_types=[
        pltpu.VMEM((2, SW, B3), jnp.int32),
        pltpu.VMEM((2, SW, B3), jnp.int32),
        pltpu.VMEM((B3, D), jnp.float32),
        pltpu.VMEM((B3, D), jnp.float32),
        pltpu.VMEM_SHARED((NACC, D), jnp.float32),
        pltpu.SemaphoreType.DMA,
        pltpu.SemaphoreType.DMA,
    ],
    compiler_params=_sc_params,
)
def _sc_propagate(
    y_ref, row_ref, col_ref, z_ref, out_ref,
    ridx, cidx, buf0, buf1, acc, gs0, gs1,
):
    s = lax.axis_index("s")
    tbase = s * RPT
    cw = C3W
    cbase = s * C3W
    nh = C3W // 2

    pltpu.sync_copy(z_ref.at[pl.ds(tbase, RPT)], acc.at[pl.ds(tbase, RPT)])

    off0 = pl.multiple_of(cbase, 8)
    pltpu.sync_copy(row_ref.at[pl.ds(off0, SW)], ridx.at[0])
    pltpu.sync_copy(col_ref.at[pl.ds(off0, SW)], cidx.at[0])
    plsc.subcore_barrier()

    def g_idx(j):
        return ridx.at[(j // SW) % 2, j % SW]

    def s_idx(j):
        return cidx.at[(j // SW) % 2, j % SW]

    def wait_g(buf, sem):
        pltpu.make_async_copy(y_ref.at[ridx.at[0, 0]], buf, sem).wait()

    pltpu.async_copy(y_ref.at[g_idx(0)], buf0, gs0)

    def body(h, carry):
        j0 = 2 * h

        @pl.when((j0 % SW == 0) & (j0 + SW < cw))
        def _():
            slot = ((j0 // SW) + 1) % 2
            off = pl.multiple_of(cbase + j0 + SW, 8)
            pltpu.sync_copy(row_ref.at[pl.ds(off, SW)], ridx.at[slot])
            pltpu.sync_copy(col_ref.at[pl.ds(off, SW)], cidx.at[slot])

        pltpu.async_copy(y_ref.at[g_idx(j0 + 1)], buf1, gs1)
        wait_g(buf0, gs0)
        pltpu.sync_copy(buf0, acc.at[s_idx(j0)], add=True)

        @pl.when(h < nh - 1)
        def _():
            pltpu.async_copy(y_ref.at[g_idx(j0 + 2)], buf0, gs0)

        wait_g(buf1, gs1)
        pltpu.sync_copy(buf1, acc.at[s_idx(j0 + 1)], add=True)
        return carry

    lax.fori_loop(0, nh, body, 0)
    plsc.subcore_barrier()

    pltpu.sync_copy(
        acc.at[pl.ds(tbase, RPT)], out_ref.at[pl.ds(tbase, RPT)]
    )


def _tc_finish_body(p_ref, y_ref, cnt_ref, self_ref, out_ref):
    acc = p_ref[...]
    cnt = jnp.sum(cnt_ref[...], axis=1)
    selfc = jnp.sum(self_ref[...], axis=1)
    loop_w = jnp.where(selfc == 0.0, 1.0, 0.0)
    deg = cnt + loop_w
    dis = lax.rsqrt(deg)
    dis = jnp.where(jnp.isinf(dis), 0.0, dis)
    out_ref[...] = dis[:, None] * (acc + loop_w[:, None] * y_ref[...])


def _tc_finish(p, y, cnt_t, self_t):
    return pl.pallas_call(
        _tc_finish_body,
        grid=(N // R2,),
        in_specs=[
            pl.BlockSpec((R2, D), lambda i: (i, 0)),
            pl.BlockSpec((R2, D), lambda i: (i, 0)),
            pl.BlockSpec((R2, NW), lambda i: (i, 0)),
            pl.BlockSpec((R2, NW), lambda i: (i, 0)),
        ],
        out_specs=pl.BlockSpec((R2, D), lambda i: (i, 0)),
        out_shape=jax.ShapeDtypeStruct((N, D), jnp.float32),
    )(p, y, cnt_t, self_t)


def kernel(x, edge_index, node_label, weight_id):
    row0 = edge_index[0]
    col0 = edge_index[1]

    cntp, selfp = _sc_bincount(row0, col0)
    cnt_t = cntp.T
    self_t = selfp.T

    masks = (
        node_label[:, None] == jnp.arange(8, dtype=node_label.dtype)[None, :]
    ).astype(x.dtype)
    y = _tc_transform(x, masks, weight_id, cnt_t, self_t)

    pad = EP - E
    rowp = jnp.concatenate(
        [row0, jnp.zeros((pad,), jnp.int32)]
    ).reshape(EP // B3, B3)
    colp = jnp.concatenate(
        [col0, jnp.full((pad,), N, jnp.int32)]
    ).reshape(EP // B3, B3)
    zfull = jnp.zeros((NACC, D), jnp.float32)

    p = _sc_propagate(y, rowp, colp, zfull)
    return _tc_finish(p, y, cnt_t, self_t)


# speedup vs baseline: 3.1148x; 3.1148x over previous
import functools

import jax
import jax.numpy as jnp
from jax import lax
from jax.experimental import pallas as pl
from jax.experimental.pallas import tpu as pltpu
from jax.experimental.pallas import tpu_sc as plsc

NC = 2
NS = 16
NW = NC * NS

N = 10000
D = 128
E = 320000

B3 = 128
C3 = 80
SW = 16
C3_0 = 128
C3_1 = 32
EP = NW * C3 * B3
RPT = 632
NACC = NS * RPT

_mesh = plsc.VectorSubcoreMesh(core_axis_name="c", subcore_axis_name="s")
_sc_params = pltpu.CompilerParams(needs_layout_passes=False)


EW1 = E // NW


@functools.partial(
    pl.kernel,
    out_type=(
        jax.ShapeDtypeStruct((NW, N), jnp.float32),
        jax.ShapeDtypeStruct((NW, N), jnp.float32),
    ),
    mesh=_mesh,
    scratch_types=[
        pltpu.VMEM((EW1,), jnp.int32),
        pltpu.VMEM((EW1,), jnp.int32),
        pltpu.VMEM((N,), jnp.float32),
        pltpu.VMEM((N,), jnp.float32),
    ],
    compiler_params=_sc_params,
)
def _sc_bincount(row_ref, col_ref, cnt_out, self_out, rowv, colv, cntl, selfl):
    c = lax.axis_index("c")
    s = lax.axis_index("s")
    w = s * NC + c
    base = w * EW1
    pltpu.sync_copy(row_ref.at[pl.ds(base, EW1)], rowv)
    pltpu.sync_copy(col_ref.at[pl.ds(base, EW1)], colv)

    zeros16 = jnp.zeros((16,), jnp.float32)

    def zbody(i, carry):
        cntl[pl.ds(i * 16, 16)] = zeros16
        selfl[pl.ds(i * 16, 16)] = zeros16
        return carry

    lax.fori_loop(0, N // 16, zbody, 0)

    ones16 = jnp.ones((16,), jnp.float32)

    def body(i, carry):
        r = rowv[pl.ds(i * 16, 16)]
        cc = colv[pl.ds(i * 16, 16)]
        plsc.addupdate_scatter(cntl, [r], ones16)
        plsc.addupdate_scatter(
            selfl, [r], jnp.where(r == cc, 1.0, 0.0).astype(jnp.float32)
        )
        return carry

    lax.fori_loop(0, EW1 // 16, body, 0)

    pltpu.sync_copy(cntl, cnt_out.at[w])
    pltpu.sync_copy(selfl, self_out.at[w])


R2 = 2000


def _tc_transform_body(x_ref, m_ref, w_ref, cnt_ref, self_ref, y_ref):
    xb = x_ref[...]
    ox = xb
    m = m_ref[...]
    for i in range(7):
        ox = ox + jnp.dot(
            xb, w_ref[i], preferred_element_type=jnp.float32
        ) * m[:, i + 1 : i + 2]
    cnt = jnp.sum(cnt_ref[...], axis=1)
    selfc = jnp.sum(self_ref[...], axis=1)
    loop_w = jnp.where(selfc == 0.0, 1.0, 0.0)
    deg = cnt + loop_w
    dis = lax.rsqrt(deg)
    dis = jnp.where(jnp.isinf(dis), 0.0, dis)
    y_ref[...] = dis[:, None] * ox


def _tc_transform(x, masks, weight_id, cnt_t, self_t):
    return pl.pallas_call(
        _tc_transform_body,
        grid=(N // R2,),
        in_specs=[
            pl.BlockSpec((R2, D), lambda i: (i, 0)),
            pl.BlockSpec((R2, 8), lambda i: (i, 0)),
            pl.BlockSpec((7, D, D), lambda i: (0, 0, 0)),
            pl.BlockSpec((R2, NW), lambda i: (i, 0)),
            pl.BlockSpec((R2, NW), lambda i: (i, 0)),
        ],
        out_specs=pl.BlockSpec((R2, D), lambda i: (i, 0)),
        out_shape=jax.ShapeDtypeStruct((N, D), jnp.float32),
    )(x, masks, weight_id, cnt_t, self_t)


@functools.partial(
    pl.kernel,
    out_type=jax.ShapeDtypeStruct((NC, NACC, D), jnp.float32),
    mesh=_mesh,
    scratch_types=[
        pltpu.VMEM((2, SW, B3), jnp.int32),
        pltpu.VMEM((2, SW, B3), jnp.int32),
        pltpu.VMEM((B3, D), jnp.float32),
        pltpu.VMEM((B3, D), jnp.float32),
        pltpu.VMEM_SHARED((NACC, D), jnp.float32),
        pltpu.SemaphoreType.DMA,
        pltpu.SemaphoreType.DMA,
    ],
    compiler_params=_sc_params,
)
def _sc_propagate(
    y_ref, row_ref, col_ref, z_ref, out_ref,
    ridx, cidx, buf0, buf1, acc, gs0, gs1,
):
    c = lax.axis_index("c")
    s = lax.axis_index("s")
    w = s * NC + c
    tbase = s * RPT
    cw = C3
    cbase = w * C3
    nh = C3 // 2

    pltpu.sync_copy(z_ref.at[pl.ds(tbase, RPT)], acc.at[pl.ds(tbase, RPT)])

    off0 = pl.multiple_of(cbase, 8)
    pltpu.sync_copy(row_ref.at[pl.ds(off0, SW)], ridx.at[0])
    pltpu.sync_copy(col_ref.at[pl.ds(off0, SW)], cidx.at[0])
    plsc.subcore_barrier()

    def g_idx(j):
        return ridx.at[(j // SW) % 2, j % SW]

    def s_idx(j):
        return cidx.at[(j // SW) % 2, j % SW]

    def wait_g(buf, sem):
        pltpu.make_async_copy(y_ref.at[ridx.at[0, 0]], buf, sem).wait()

    pltpu.async_copy(y_ref.at[g_idx(0)], buf0, gs0)

    def body(h, carry):
        j0 = 2 * h

        @pl.when((j0 % SW == 0) & (j0 + SW < cw))
        def _():
            slot = ((j0 // SW) + 1) % 2
            off = pl.multiple_of(cbase + j0 + SW, 8)
            pltpu.sync_copy(row_ref.at[pl.ds(off, SW)], ridx.at[slot])
            pltpu.sync_copy(col_ref.at[pl.ds(off, SW)], cidx.at[slot])

        pltpu.async_copy(y_ref.at[g_idx(j0 + 1)], buf1, gs1)
        wait_g(buf0, gs0)
        pltpu.sync_copy(buf0, acc.at[s_idx(j0)], add=True)

        @pl.when(h < nh - 1)
        def _():
            pltpu.async_copy(y_ref.at[g_idx(j0 + 2)], buf0, gs0)

        wait_g(buf1, gs1)
        pltpu.sync_copy(buf1, acc.at[s_idx(j0 + 1)], add=True)
        return carry

    lax.fori_loop(0, nh, body, 0)
    plsc.subcore_barrier()

    pltpu.sync_copy(
        acc.at[pl.ds(tbase, RPT)], out_ref.at[c, pl.ds(tbase, RPT)]
    )


def _tc_finish_body(p_ref, y_ref, cnt_ref, self_ref, out_ref):
    acc = p_ref[0] + p_ref[1]
    cnt = jnp.sum(cnt_ref[...], axis=1)
    selfc = jnp.sum(self_ref[...], axis=1)
    loop_w = jnp.where(selfc == 0.0, 1.0, 0.0)
    deg = cnt + loop_w
    dis = lax.rsqrt(deg)
    dis = jnp.where(jnp.isinf(dis), 0.0, dis)
    out_ref[...] = dis[:, None] * (acc + loop_w[:, None] * y_ref[...])


def _tc_finish(p, y, cnt_t, self_t):
    return pl.pallas_call(
        _tc_finish_body,
        grid=(N // R2,),
        in_specs=[
            pl.BlockSpec((NC, R2, D), lambda i: (0, i, 0)),
            pl.BlockSpec((R2, D), lambda i: (i, 0)),
            pl.BlockSpec((R2, NW), lambda i: (i, 0)),
            pl.BlockSpec((R2, NW), lambda i: (i, 0)),
        ],
        out_specs=pl.BlockSpec((R2, D), lambda i: (i, 0)),
        out_shape=jax.ShapeDtypeStruct((N, D), jnp.float32),
    )(p, y, cnt_t, self_t)


def kernel(x, edge_index, node_label, weight_id):
    row0 = edge_index[0]
    col0 = edge_index[1]

    cntp, selfp = _sc_bincount(row0, col0)
    cnt_t = cntp.T
    self_t = selfp.T

    masks = (
        node_label[:, None] == jnp.arange(8, dtype=node_label.dtype)[None, :]
    ).astype(x.dtype)
    y = _tc_transform(x, masks, weight_id, cnt_t, self_t)

    pad = EP - E
    ar = jnp.arange(pad, dtype=jnp.int32)
    rowp = jnp.concatenate([row0, ar % N]).reshape(EP // B3, B3)
    colp = jnp.concatenate(
        [col0, N + (ar % (NACC - N))]
    ).reshape(EP // B3, B3)
    zfull = jnp.zeros((NACC, D), jnp.float32)

    p = _sc_propagate(y, rowp, colp, zfull)
    return _tc_finish(p, y, cnt_t, self_t)


# speedup vs baseline: 3.1698x; 1.0177x over previous
import functools

import jax
import jax.numpy as jnp
from jax import lax
from jax.experimental import pallas as pl
from jax.experimental.pallas import tpu as pltpu
from jax.experimental.pallas import tpu_sc as plsc

NC = 2
NS = 16
NW = NC * NS

N = 10000
D = 128
E = 320000

B3 = 128
C3 = 80
SW = 16
C3_0 = 128
C3_1 = 32
EP = NW * C3 * B3
RPT = 632
NACC = NS * RPT

_mesh = plsc.VectorSubcoreMesh(core_axis_name="c", subcore_axis_name="s")
_sc_params = pltpu.CompilerParams(needs_layout_passes=False)


EW1 = E // NW


@functools.partial(
    pl.kernel,
    out_type=(
        jax.ShapeDtypeStruct((NW, N), jnp.float32),
        jax.ShapeDtypeStruct((NW, N), jnp.float32),
    ),
    mesh=_mesh,
    scratch_types=[
        pltpu.VMEM((EW1,), jnp.int32),
        pltpu.VMEM((EW1,), jnp.int32),
        pltpu.VMEM((N,), jnp.float32),
        pltpu.VMEM((N,), jnp.float32),
    ],
    compiler_params=_sc_params,
)
def _sc_bincount(row_ref, col_ref, cnt_out, self_out, rowv, colv, cntl, selfl):
    c = lax.axis_index("c")
    s = lax.axis_index("s")
    w = s * NC + c
    base = w * EW1
    pltpu.sync_copy(row_ref.at[pl.ds(base, EW1)], rowv)
    pltpu.sync_copy(col_ref.at[pl.ds(base, EW1)], colv)

    zeros16 = jnp.zeros((16,), jnp.float32)

    def zbody(i, carry):
        cntl[pl.ds(i * 16, 16)] = zeros16
        selfl[pl.ds(i * 16, 16)] = zeros16
        return carry

    lax.fori_loop(0, N // 16, zbody, 0)

    ones16 = jnp.ones((16,), jnp.float32)

    def body(i, carry):
        r = rowv[pl.ds(i * 16, 16)]
        cc = colv[pl.ds(i * 16, 16)]
        plsc.addupdate_scatter(cntl, [r], ones16)
        plsc.addupdate_scatter(
            selfl, [r], jnp.where(r == cc, 1.0, 0.0).astype(jnp.float32)
        )
        return carry

    lax.fori_loop(0, EW1 // 16, body, 0)

    pltpu.sync_copy(cntl, cnt_out.at[w])
    pltpu.sync_copy(selfl, self_out.at[w])


R2 = 2000


def _tc_transform_body(x_ref, m_ref, w_ref, cnt_ref, self_ref, y_ref):
    xb = x_ref[...]
    ox = xb
    m = m_ref[...]
    xb16 = xb.astype(jnp.bfloat16)
    for i in range(7):
        ox = ox + jnp.dot(
            xb16,
            w_ref[i].astype(jnp.bfloat16),
            preferred_element_type=jnp.float32,
        ) * m[:, i + 1 : i + 2]
    cnt = jnp.sum(cnt_ref[...], axis=1)
    selfc = jnp.sum(self_ref[...], axis=1)
    loop_w = jnp.where(selfc == 0.0, 1.0, 0.0)
    deg = cnt + loop_w
    dis = lax.rsqrt(deg)
    dis = jnp.where(jnp.isinf(dis), 0.0, dis)
    y_ref[...] = dis[:, None] * ox


def _tc_transform(x, masks, weight_id, cnt_t, self_t):
    return pl.pallas_call(
        _tc_transform_body,
        grid=(N // R2,),
        in_specs=[
            pl.BlockSpec((R2, D), lambda i: (i, 0)),
            pl.BlockSpec((R2, 8), lambda i: (i, 0)),
            pl.BlockSpec((7, D, D), lambda i: (0, 0, 0)),
            pl.BlockSpec((R2, NW), lambda i: (i, 0)),
            pl.BlockSpec((R2, NW), lambda i: (i, 0)),
        ],
        out_specs=pl.BlockSpec((R2, D), lambda i: (i, 0)),
        out_shape=jax.ShapeDtypeStruct((N, D), jnp.float32),
    )(x, masks, weight_id, cnt_t, self_t)


@functools.partial(
    pl.kernel,
    out_type=jax.ShapeDtypeStruct((NC, NACC, D), jnp.float32),
    mesh=_mesh,
    scratch_types=[
        pltpu.VMEM((2, SW, B3), jnp.int32),
        pltpu.VMEM((2, SW, B3), jnp.int32),
        pltpu.VMEM((B3, D), jnp.float32),
        pltpu.VMEM((B3, D), jnp.float32),
        pltpu.VMEM_SHARED((NACC, D), jnp.float32),
        pltpu.SemaphoreType.DMA,
        pltpu.SemaphoreType.DMA,
        pltpu.SemaphoreType.DMA,
    ],
    compiler_params=_sc_params,
)
def _sc_propagate(
    y_ref, row_ref, col_ref, z_ref, out_ref,
    ridx, cidx, buf0, buf1, acc, gs0, gs1, xs,
):
    c = lax.axis_index("c")
    s = lax.axis_index("s")
    w = s * NC + c
    tbase = s * RPT
    cw = C3
    cbase = w * C3
    nh = C3 // 2

    pltpu.sync_copy(z_ref.at[pl.ds(tbase, RPT)], acc.at[pl.ds(tbase, RPT)])

    off0 = pl.multiple_of(cbase, 8)
    pltpu.sync_copy(row_ref.at[pl.ds(off0, SW)], ridx.at[0])
    pltpu.sync_copy(col_ref.at[pl.ds(off0, SW)], cidx.at[0])
    plsc.subcore_barrier()

    def g_idx(j):
        return ridx.at[(j // SW) % 2, j % SW]

    def s_idx(j):
        return cidx.at[(j // SW) % 2, j % SW]

    def wait_g(buf, sem):
        pltpu.make_async_copy(y_ref.at[ridx.at[0, 0]], buf, sem).wait()

    pltpu.async_copy(y_ref.at[g_idx(0)], buf0, gs0)

    def body(h, carry):
        j0 = 2 * h

        @pl.when((j0 % SW == 0) & (j0 + SW < cw))
        def _():
            slot = ((j0 // SW) + 1) % 2
            off = pl.multiple_of(cbase + j0 + SW, 8)
            pltpu.async_copy(row_ref.at[pl.ds(off, SW)], ridx.at[slot], xs)
            pltpu.async_copy(col_ref.at[pl.ds(off, SW)], cidx.at[slot], xs)

        @pl.when((j0 % SW == SW - 2) & (j0 + 2 < cw))
        def _():
            pltpu.make_async_copy(
                row_ref.at[pl.ds(0, SW)], ridx.at[0], xs
            ).wait()
            pltpu.make_async_copy(
                col_ref.at[pl.ds(0, SW)], cidx.at[0], xs
            ).wait()

        pltpu.async_copy(y_ref.at[g_idx(j0 + 1)], buf1, gs1)
        wait_g(buf0, gs0)
        pltpu.sync_copy(buf0, acc.at[s_idx(j0)], add=True)

        @pl.when(h < nh - 1)
        def _():
            pltpu.async_copy(y_ref.at[g_idx(j0 + 2)], buf0, gs0)

        wait_g(buf1, gs1)
        pltpu.sync_copy(buf1, acc.at[s_idx(j0 + 1)], add=True)
        return carry

    lax.fori_loop(0, nh, body, 0)
    plsc.subcore_barrier()

    pltpu.sync_copy(
        acc.at[pl.ds(tbase, RPT)], out_ref.at[c, pl.ds(tbase, RPT)]
    )


def _tc_finish_body(p_ref, y_ref, cnt_ref, self_ref, out_ref):
    acc = p_ref[0] + p_ref[1]
    cnt = jnp.sum(cnt_ref[...], axis=1)
    selfc = jnp.sum(self_ref[...], axis=1)
    loop_w = jnp.where(selfc == 0.0, 1.0, 0.0)
    deg = cnt + loop_w
    dis = lax.rsqrt(deg)
    dis = jnp.where(jnp.isinf(dis), 0.0, dis)
    out_ref[...] = dis[:, None] * (acc + loop_w[:, None] * y_ref[...])


def _tc_finish(p, y, cnt_t, self_t):
    return pl.pallas_call(
        _tc_finish_body,
        grid=(N // R2,),
        in_specs=[
            pl.BlockSpec((NC, R2, D), lambda i: (0, i, 0)),
            pl.BlockSpec((R2, D), lambda i: (i, 0)),
            pl.BlockSpec((R2, NW), lambda i: (i, 0)),
            pl.BlockSpec((R2, NW), lambda i: (i, 0)),
        ],
        out_specs=pl.BlockSpec((R2, D), lambda i: (i, 0)),
        out_shape=jax.ShapeDtypeStruct((N, D), jnp.float32),
    )(p, y, cnt_t, self_t)


def kernel(x, edge_index, node_label, weight_id):
    row0 = edge_index[0]
    col0 = edge_index[1]

    cntp, selfp = _sc_bincount(row0, col0)
    cnt_t = cntp.T
    self_t = selfp.T

    masks = (
        node_label[:, None] == jnp.arange(8, dtype=node_label.dtype)[None, :]
    ).astype(x.dtype)
    y = _tc_transform(x, masks, weight_id, cnt_t, self_t)

    pad = EP - E
    ar = jnp.arange(pad, dtype=jnp.int32)
    rowp = jnp.concatenate([row0, ar % N]).reshape(EP // B3, B3)
    colp = jnp.concatenate(
        [col0, N + (ar % (NACC - N))]
    ).reshape(EP // B3, B3)
    zfull = jnp.zeros((NACC, D), jnp.float32)

    p = _sc_propagate(y, rowp, colp, zfull)
    return _tc_finish(p, y, cnt_t, self_t)


# speedup vs baseline: 3.1719x; 1.0007x over previous
import functools

import jax
import jax.numpy as jnp
from jax import lax
from jax.experimental import pallas as pl
from jax.experimental.pallas import tpu as pltpu
from jax.experimental.pallas import tpu_sc as plsc

NC = 2
NS = 16
NW = NC * NS

N = 10000
D = 128
E = 320000

B3 = 128
C3 = 80
SW = 16
C3_0 = 128
C3_1 = 32
EP = NW * C3 * B3
RPT = 632
NACC = NS * RPT

_mesh = plsc.VectorSubcoreMesh(core_axis_name="c", subcore_axis_name="s")
_sc_params = pltpu.CompilerParams(needs_layout_passes=False)


EW1 = E // NW


@functools.partial(
    pl.kernel,
    out_type=(
        jax.ShapeDtypeStruct((NW, N), jnp.float32),
        jax.ShapeDtypeStruct((NW, N), jnp.float32),
    ),
    mesh=_mesh,
    scratch_types=[
        pltpu.VMEM((EW1,), jnp.int32),
        pltpu.VMEM((EW1,), jnp.int32),
        pltpu.VMEM((N,), jnp.float32),
        pltpu.VMEM((N,), jnp.float32),
    ],
    compiler_params=_sc_params,
)
def _sc_bincount(row_ref, col_ref, cnt_out, self_out, rowv, colv, cntl, selfl):
    c = lax.axis_index("c")
    s = lax.axis_index("s")
    w = s * NC + c
    base = w * EW1
    pltpu.sync_copy(row_ref.at[pl.ds(base, EW1)], rowv)
    pltpu.sync_copy(col_ref.at[pl.ds(base, EW1)], colv)

    zeros16 = jnp.zeros((16,), jnp.float32)

    def zbody(i, carry):
        cntl[pl.ds(i * 16, 16)] = zeros16
        selfl[pl.ds(i * 16, 16)] = zeros16
        return carry

    lax.fori_loop(0, N // 16, zbody, 0)

    ones16 = jnp.ones((16,), jnp.float32)

    def body(i, carry):
        r = rowv[pl.ds(i * 16, 16)]
        cc = colv[pl.ds(i * 16, 16)]
        plsc.addupdate_scatter(cntl, [r], ones16)
        plsc.addupdate_scatter(
            selfl, [r], jnp.where(r == cc, 1.0, 0.0).astype(jnp.float32)
        )
        return carry

    lax.fori_loop(0, EW1 // 16, body, 0)

    pltpu.sync_copy(cntl, cnt_out.at[w])
    pltpu.sync_copy(selfl, self_out.at[w])


R2 = 2000


def _tc_transform_body(x_ref, m_ref, w_ref, cnt_ref, self_ref, y_ref):
    xb = x_ref[...]
    ox = xb
    m = m_ref[...]
    xb16 = xb.astype(jnp.bfloat16)
    for i in range(7):
        ox = ox + jnp.dot(
            xb16,
            w_ref[i].astype(jnp.bfloat16),
            preferred_element_type=jnp.float32,
        ) * m[:, i + 1 : i + 2]
    cnt = jnp.sum(cnt_ref[...], axis=1)
    selfc = jnp.sum(self_ref[...], axis=1)
    loop_w = jnp.where(selfc == 0.0, 1.0, 0.0)
    deg = cnt + loop_w
    dis = lax.rsqrt(deg)
    dis = jnp.where(jnp.isinf(dis), 0.0, dis)
    y_ref[...] = dis[:, None] * ox


def _tc_transform(x, masks, weight_id, cnt_t, self_t):
    return pl.pallas_call(
        _tc_transform_body,
        grid=(N // R2,),
        in_specs=[
            pl.BlockSpec((R2, D), lambda i: (i, 0)),
            pl.BlockSpec((R2, 8), lambda i: (i, 0)),
            pl.BlockSpec((7, D, D), lambda i: (0, 0, 0)),
            pl.BlockSpec((R2, NW), lambda i: (i, 0)),
            pl.BlockSpec((R2, NW), lambda i: (i, 0)),
        ],
        out_specs=pl.BlockSpec((R2, D), lambda i: (i, 0)),
        out_shape=jax.ShapeDtypeStruct((N, D), jnp.float32),
    )(x, masks, weight_id, cnt_t, self_t)


@functools.partial(
    pl.kernel,
    out_type=jax.ShapeDtypeStruct((NC, NACC, D), jnp.float32),
    mesh=_mesh,
    scratch_types=[
        pltpu.VMEM((2, SW, B3), jnp.int32),
        pltpu.VMEM((2, SW, B3), jnp.int32),
        pltpu.VMEM((B3, D), jnp.float32),
        pltpu.VMEM((B3, D), jnp.float32),
        pltpu.VMEM_SHARED((NACC, D), jnp.float32),
        pltpu.SemaphoreType.DMA,
        pltpu.SemaphoreType.DMA,
        pltpu.SemaphoreType.DMA,
    ],
    compiler_params=_sc_params,
)
def _sc_propagate(
    y_ref, row_ref, col_ref, z_ref, out_ref,
    ridx, cidx, buf0, buf1, acc, gs0, gs1, xs,
):
    c = lax.axis_index("c")
    s = lax.axis_index("s")
    w = s * NC + c
    tbase = s * RPT
    cw = C3
    cbase = w * C3
    nh = C3 // 2

    pltpu.sync_copy(z_ref.at[pl.ds(tbase, RPT)], acc.at[pl.ds(tbase, RPT)])

    off0 = pl.multiple_of(cbase, 8)
    pltpu.sync_copy(row_ref.at[pl.ds(off0, SW)], ridx.at[0])
    pltpu.sync_copy(col_ref.at[pl.ds(off0, SW)], cidx.at[0])
    plsc.subcore_barrier()

    def g_idx(j):
        return ridx.at[(j // SW) % 2, j % SW]

    def s_idx(j):
        return cidx.at[(j // SW) % 2, j % SW]

    def wait_g(buf, sem):
        pltpu.make_async_copy(y_ref.at[ridx.at[0, 0]], buf, sem).wait()

    pltpu.async_copy(y_ref.at[g_idx(0)], buf0, gs0)

    def body(h, carry):
        j0 = 2 * h

        @pl.when((j0 % SW == 0) & (j0 + SW < cw))
        def _():
            slot = ((j0 // SW) + 1) % 2
            off = pl.multiple_of(cbase + j0 + SW, 8)
            pltpu.async_copy(row_ref.at[pl.ds(off, SW)], ridx.at[slot], xs)
            pltpu.async_copy(col_ref.at[pl.ds(off, SW)], cidx.at[slot], xs)

        @pl.when((j0 % SW == SW - 2) & (j0 + 2 < cw))
        def _():
            pltpu.make_async_copy(
                row_ref.at[pl.ds(0, SW)], ridx.at[0], xs
            ).wait()
            pltpu.make_async_copy(
                col_ref.at[pl.ds(0, SW)], cidx.at[0], xs
            ).wait()

        pltpu.async_copy(y_ref.at[g_idx(j0 + 1)], buf1, gs1)
        wait_g(buf0, gs0)
        pltpu.sync_copy(buf0, acc.at[s_idx(j0)], add=True)

        @pl.when(h < nh - 1)
        def _():
            pltpu.async_copy(y_ref.at[g_idx(j0 + 2)], buf0, gs0)

        wait_g(buf1, gs1)
        pltpu.sync_copy(buf1, acc.at[s_idx(j0 + 1)], add=True)
        return carry

    lax.fori_loop(0, nh, body, 0)
    plsc.subcore_barrier()

    pltpu.sync_copy(
        acc.at[pl.ds(tbase, RPT)], out_ref.at[c, pl.ds(tbase, RPT)]
    )


def _tc_finish_body(p_ref, y_ref, cnt_ref, self_ref, out_ref):
    acc = p_ref[0] + p_ref[1]
    cnt = jnp.sum(cnt_ref[...], axis=1)
    selfc = jnp.sum(self_ref[...], axis=1)
    loop_w = jnp.where(selfc == 0.0, 1.0, 0.0)
    deg = cnt + loop_w
    dis = lax.rsqrt(deg)
    dis = jnp.where(jnp.isinf(dis), 0.0, dis)
    out_ref[...] = dis[:, None] * (acc + loop_w[:, None] * y_ref[...])


def _tc_finish(p, y, cnt_t, self_t):
    return pl.pallas_call(
        _tc_finish_body,
        grid=(N // R2,),
        in_specs=[
            pl.BlockSpec((NC, R2, D), lambda i: (0, i, 0)),
            pl.BlockSpec((R2, D), lambda i: (i, 0)),
            pl.BlockSpec((R2, NW), lambda i: (i, 0)),
            pl.BlockSpec((R2, NW), lambda i: (i, 0)),
        ],
        out_specs=pl.BlockSpec((R2, D), lambda i: (i, 0)),
        out_shape=jax.ShapeDtypeStruct((N, D), jnp.float32),
    )(p, y, cnt_t, self_t)


def kernel(x, edge_index, node_label, weight_id):
    row0 = edge_index[0]
    col0 = edge_index[1]

    cntp, selfp = _sc_bincount(row0, col0)
    cnt_t = cntp.T
    self_t = selfp.T

    masks = (
        node_label[:, None] == jnp.arange(8, dtype=node_label.dtype)[None, :]
    ).astype(x.dtype)
    y = _tc_transform(x, masks, weight_id, cnt_t, self_t)

    pad = EP - E
    ar = jnp.arange(pad, dtype=jnp.int32)
    rowp = jnp.concatenate([row0, ar & 8191]).reshape(EP // B3, B3)
    colp = jnp.concatenate([col0, N + (ar & 63)]).reshape(EP // B3, B3)
    zfull = jnp.zeros((NACC, D), jnp.float32)

    p = _sc_propagate(y, rowp, colp, zfull)
    return _tc_finish(p, y, cnt_t, self_t)


# speedup vs baseline: 3.1735x; 1.0005x over previous
import functools

import jax
import jax.numpy as jnp
from jax import lax
from jax.experimental import pallas as pl
from jax.experimental.pallas import tpu as pltpu
from jax.experimental.pallas import tpu_sc as plsc

NC = 2
NS = 16
NW = NC * NS

N = 10000
D = 128
E = 320000

B3 = 128
C3 = 80
SW = 16
EP = NW * C3 * B3
RPT = 632
NACC = NS * RPT

_mesh = plsc.VectorSubcoreMesh(core_axis_name="c", subcore_axis_name="s")
_sc_params = pltpu.CompilerParams(needs_layout_passes=False)


EW1 = E // NW


@functools.partial(
    pl.kernel,
    out_type=(
        jax.ShapeDtypeStruct((NW, N), jnp.float32),
        jax.ShapeDtypeStruct((NW, N), jnp.float32),
    ),
    mesh=_mesh,
    scratch_types=[
        pltpu.VMEM((EW1,), jnp.int32),
        pltpu.VMEM((EW1,), jnp.int32),
        pltpu.VMEM((N,), jnp.float32),
        pltpu.VMEM((N,), jnp.float32),
    ],
    compiler_params=_sc_params,
)
def _sc_bincount(row_ref, col_ref, cnt_out, self_out, rowv, colv, cntl, selfl):
    c = lax.axis_index("c")
    s = lax.axis_index("s")
    w = s * NC + c
    base = w * EW1
    pltpu.sync_copy(row_ref.at[pl.ds(base, EW1)], rowv)
    pltpu.sync_copy(col_ref.at[pl.ds(base, EW1)], colv)

    zeros16 = jnp.zeros((16,), jnp.float32)

    def zbody(i, carry):
        cntl[pl.ds(i * 16, 16)] = zeros16
        selfl[pl.ds(i * 16, 16)] = zeros16
        return carry

    lax.fori_loop(0, N // 16, zbody, 0)

    ones16 = jnp.ones((16,), jnp.float32)

    def body(i, carry):
        r = rowv[pl.ds(i * 16, 16)]
        cc = colv[pl.ds(i * 16, 16)]
        plsc.addupdate_scatter(cntl, [r], ones16)
        plsc.addupdate_scatter(
            selfl, [r], jnp.where(r == cc, 1.0, 0.0).astype(jnp.float32)
        )
        return carry

    lax.fori_loop(0, EW1 // 16, body, 0)

    pltpu.sync_copy(cntl, cnt_out.at[w])
    pltpu.sync_copy(selfl, self_out.at[w])


R2 = 2000


def _tc_transform_body(x_ref, m_ref, w_ref, cnt_ref, self_ref, y_ref):
    xb = x_ref[...]
    ox = xb
    m = m_ref[...]
    xb16 = xb.astype(jnp.bfloat16)
    for i in range(7):
        ox = ox + jnp.dot(
            xb16,
            w_ref[i].astype(jnp.bfloat16),
            preferred_element_type=jnp.float32,
        ) * m[:, i + 1 : i + 2]
    cnt = jnp.sum(cnt_ref[...], axis=1)
    selfc = jnp.sum(self_ref[...], axis=1)
    loop_w = jnp.where(selfc == 0.0, 1.0, 0.0)
    deg = cnt + loop_w
    dis = lax.rsqrt(deg)
    dis = jnp.where(jnp.isinf(dis), 0.0, dis)
    y_ref[...] = dis[:, None] * ox


def _tc_transform(x, masks, weight_id, cnt_t, self_t):
    return pl.pallas_call(
        _tc_transform_body,
        grid=(N // R2,),
        in_specs=[
            pl.BlockSpec((R2, D), lambda i: (i, 0)),
            pl.BlockSpec((R2, 8), lambda i: (i, 0)),
            pl.BlockSpec((7, D, D), lambda i: (0, 0, 0)),
            pl.BlockSpec((R2, NW), lambda i: (i, 0)),
            pl.BlockSpec((R2, NW), lambda i: (i, 0)),
        ],
        out_specs=pl.BlockSpec((R2, D), lambda i: (i, 0)),
        out_shape=jax.ShapeDtypeStruct((N, D), jnp.float32),
    )(x, masks, weight_id, cnt_t, self_t)


@functools.partial(
    pl.kernel,
    out_type=jax.ShapeDtypeStruct((NC, NACC, D), jnp.float32),
    mesh=_mesh,
    scratch_types=[
        pltpu.VMEM((2, SW, B3), jnp.int32),
        pltpu.VMEM((2, SW, B3), jnp.int32),
        pltpu.VMEM((B3, D), jnp.float32),
        pltpu.VMEM((B3, D), jnp.float32),
        pltpu.VMEM_SHARED((NACC, D), jnp.float32),
        pltpu.SemaphoreType.DMA,
        pltpu.SemaphoreType.DMA,
        pltpu.SemaphoreType.DMA,
    ],
    compiler_params=_sc_params,
)
def _sc_propagate(
    y_ref, row_ref, col_ref, z_ref, out_ref,
    ridx, cidx, buf0, buf1, acc, gs0, gs1, xs,
):
    c = lax.axis_index("c")
    s = lax.axis_index("s")
    w = s * NC + c
    tbase = s * RPT
    cw = C3
    cbase = w * C3
    nh = C3 // 2

    pltpu.sync_copy(z_ref.at[pl.ds(tbase, RPT)], acc.at[pl.ds(tbase, RPT)])

    off0 = pl.multiple_of(cbase, 8)
    pltpu.sync_copy(row_ref.at[pl.ds(off0, SW)], ridx.at[0])
    pltpu.sync_copy(col_ref.at[pl.ds(off0, SW)], cidx.at[0])
    plsc.subcore_barrier()

    def g_idx(j):
        return ridx.at[(j // SW) % 2, j % SW]

    def s_idx(j):
        return cidx.at[(j // SW) % 2, j % SW]

    def wait_g(buf, sem):
        pltpu.make_async_copy(y_ref.at[ridx.at[0, 0]], buf, sem).wait()

    pltpu.async_copy(y_ref.at[g_idx(0)], buf0, gs0)

    def body(h, carry):
        j0 = 2 * h

        @pl.when((j0 % SW == 0) & (j0 + SW < cw))
        def _():
            slot = ((j0 // SW) + 1) % 2
            off = pl.multiple_of(cbase + j0 + SW, 8)
            pltpu.async_copy(row_ref.at[pl.ds(off, SW)], ridx.at[slot], xs)
            pltpu.async_copy(col_ref.at[pl.ds(off, SW)], cidx.at[slot], xs)

        @pl.when((j0 % SW == SW - 2) & (j0 + 2 < cw))
        def _():
            pltpu.make_async_copy(
                row_ref.at[pl.ds(0, SW)], ridx.at[0], xs
            ).wait()
            pltpu.make_async_copy(
                col_ref.at[pl.ds(0, SW)], cidx.at[0], xs
            ).wait()

        pltpu.async_copy(y_ref.at[g_idx(j0 + 1)], buf1, gs1)
        wait_g(buf0, gs0)
        pltpu.sync_copy(buf0, acc.at[s_idx(j0)], add=True)

        @pl.when(h < nh - 1)
        def _():
            pltpu.async_copy(y_ref.at[g_idx(j0 + 2)], buf0, gs0)

        wait_g(buf1, gs1)
        pltpu.sync_copy(buf1, acc.at[s_idx(j0 + 1)], add=True)
        return carry

    lax.fori_loop(0, nh, body, 0)
    plsc.subcore_barrier()

    pltpu.sync_copy(
        acc.at[pl.ds(tbase, RPT)], out_ref.at[c, pl.ds(tbase, RPT)]
    )


def _tc_finish_body(p_ref, y_ref, cnt_ref, self_ref, out_ref):
    acc = p_ref[0] + p_ref[1]
    cnt = jnp.sum(cnt_ref[...], axis=1)
    selfc = jnp.sum(self_ref[...], axis=1)
    loop_w = jnp.where(selfc == 0.0, 1.0, 0.0)
    deg = cnt + loop_w
    dis = lax.rsqrt(deg)
    dis = jnp.where(jnp.isinf(dis), 0.0, dis)
    out_ref[...] = dis[:, None] * (acc + loop_w[:, None] * y_ref[...])


def _tc_finish(p, y, cnt_t, self_t):
    return pl.pallas_call(
        _tc_finish_body,
        grid=(N // R2,),
        in_specs=[
            pl.BlockSpec((NC, R2, D), lambda i: (0, i, 0)),
            pl.BlockSpec((R2, D), lambda i: (i, 0)),
            pl.BlockSpec((R2, NW), lambda i: (i, 0)),
            pl.BlockSpec((R2, NW), lambda i: (i, 0)),
        ],
        out_specs=pl.BlockSpec((R2, D), lambda i: (i, 0)),
        out_shape=jax.ShapeDtypeStruct((N, D), jnp.float32),
    )(p, y, cnt_t, self_t)


def kernel(x, edge_index, node_label, weight_id):
    row0 = edge_index[0]
    col0 = edge_index[1]

    cntp, selfp = _sc_bincount(row0, col0)
    cnt_t = cntp.T
    self_t = selfp.T

    masks = (
        node_label[:, None] == jnp.arange(8, dtype=node_label.dtype)[None, :]
    ).astype(x.dtype)
    y = _tc_transform(x, masks, weight_id, cnt_t, self_t)

    pad = EP - E
    ar = jnp.arange(pad, dtype=jnp.int32)
    rowp = jnp.concatenate([row0, ar & 8191]).reshape(EP // B3, B3)
    colp = jnp.concatenate([col0, N + (ar & 63)]).reshape(EP // B3, B3)
    zfull = jnp.zeros((NACC, D), jnp.float32)

    p = _sc_propagate(y, rowp, colp, zfull)
    return _tc_finish(p, y, cnt_t, self_t)
